# baseline (speedup 1.0000x reference)
import jax
import jax.numpy as jnp
import numpy as np
from jax.experimental import pallas as pl

SCORE_THRESH = 0.05
NMS_THRESH = 0.5
DET_PER_IMG = 100
PRE_NMS_TOPK = 300
IMG_W = 1024.0
IMG_H = 1024.0
BBOX_XFORM_CLIP = float(np.log(1000.0 / 16.0))
K_PAD = 512
TILE = 2000


def _dense_kernel(logits_ref, dx_ref, dy_ref, dw_ref, dh_ref,
                  px1_ref, py1_ref, px2_ref, py2_ref,
                  prob_ref, x1_ref, y1_ref, x2_ref, y2_ref):
    logits = logits_ref[...]
    m = jnp.max(logits, axis=1, keepdims=True)
    e = jnp.exp(logits - m)
    prob_ref[...] = e / jnp.sum(e, axis=1, keepdims=True)

    px1 = px1_ref[...]
    py1 = py1_ref[...]
    widths = px2_ref[...] - px1
    heights = py2_ref[...] - py1
    ctr_x = px1 + 0.5 * widths
    ctr_y = py1 + 0.5 * heights

    dx = dx_ref[...] / 10.0
    dy = dy_ref[...] / 10.0
    dw = jnp.minimum(dw_ref[...] / 5.0, BBOX_XFORM_CLIP)
    dh = jnp.minimum(dh_ref[...] / 5.0, BBOX_XFORM_CLIP)
    pcx = dx * widths + ctr_x
    pcy = dy * heights + ctr_y
    pw = jnp.exp(dw) * widths
    ph = jnp.exp(dh) * heights
    x1_ref[...] = jnp.clip(pcx - 0.5 * pw, 0.0, IMG_W - 1.0)
    y1_ref[...] = jnp.clip(pcy - 0.5 * ph, 0.0, IMG_H - 1.0)
    x2_ref[...] = jnp.clip(pcx + 0.5 * pw, 0.0, IMG_W - 1.0)
    y2_ref[...] = jnp.clip(pcy + 0.5 * ph, 0.0, IMG_H - 1.0)


def _nms_kernel(x1_ref, y1_ref, x2_ref, y2_ref,
                x1c_ref, y1c_ref, x2c_ref, y2c_ref,
                sc_ref, out_ref):
    x1 = x1_ref[0]
    y1 = y1_ref[0]
    x2 = x2_ref[0]
    y2 = y2_ref[0]
    x1c = x1c_ref[0]
    y1c = y1c_ref[0]
    x2c = x2c_ref[0]
    y2c = y2c_ref[0]
    sc = sc_ref[0]

    area = (x2 - x1 + 1.0) * (y2 - y1 + 1.0)
    areac = (x2c - x1c + 1.0) * (y2c - y1c + 1.0)
    w = jnp.clip(jnp.minimum(x2c, x2) - jnp.maximum(x1c, x1) + 1.0, 0.0, None)
    h = jnp.clip(jnp.minimum(y2c, y2) - jnp.maximum(y1c, y1) + 1.0, 0.0, None)
    inter = w * h
    iou = inter / (areac + area - inter + 1e-9)

    ar = jax.lax.broadcasted_iota(jnp.int32, (1, K_PAD), 1)
    valid = jnp.where(sc > SCORE_THRESH, 1.0, 0.0)

    def body(i, keep):
        onehot = (ar == i).astype(jnp.float32)
        iou_i = jnp.dot(onehot, iou, preferred_element_type=jnp.float32)
        keep_i = jnp.sum(onehot * keep)
        sup = jnp.where((iou_i > NMS_THRESH) & (ar > i), keep_i, 0.0)
        return keep * (1.0 - sup)

    keep = jax.lax.fori_loop(0, PRE_NMS_TOPK, body, valid)
    out_ref[0] = keep * sc


def kernel(class_logits, box_regression, proposals):
    N, C = class_logits.shape
    deltas = box_regression.reshape(N, C, 4)
    dx = deltas[..., 0]
    dy = deltas[..., 1]
    dw = deltas[..., 2]
    dh = deltas[..., 3]
    px1 = proposals[:, 0:1]
    py1 = proposals[:, 1:2]
    px2 = proposals[:, 2:3]
    py2 = proposals[:, 3:4]

    grid = (N // TILE,)
    wide = pl.BlockSpec((TILE, C), lambda i: (i, 0))
    thin = pl.BlockSpec((TILE, 1), lambda i: (i, 0))
    outs = [jax.ShapeDtypeStruct((N, C), jnp.float32) for _ in range(5)]
    prob, x1, y1, x2, y2 = pl.pallas_call(
        _dense_kernel,
        grid=grid,
        in_specs=[wide, wide, wide, wide, wide, thin, thin, thin, thin],
        out_specs=[wide] * 5,
        out_shape=outs,
    )(class_logits, dx, dy, dw, dh, px1, py1, px2, py2)

    scores_pc = prob[:, 1:].T
    sc, idx = jax.lax.top_k(scores_pc, PRE_NMS_TOPK)
    bx1 = jnp.take_along_axis(x1[:, 1:].T, idx, axis=1)
    by1 = jnp.take_along_axis(y1[:, 1:].T, idx, axis=1)
    bx2 = jnp.take_along_axis(x2[:, 1:].T, idx, axis=1)
    by2 = jnp.take_along_axis(y2[:, 1:].T, idx, axis=1)

    pad = ((0, 0), (0, K_PAD - PRE_NMS_TOPK))
    bx1p = jnp.pad(bx1, pad)
    by1p = jnp.pad(by1, pad)
    bx2p = jnp.pad(bx2, pad)
    by2p = jnp.pad(by2, pad)
    scp = jnp.pad(sc, pad)

    nclass = C - 1
    row = pl.BlockSpec((1, 1, K_PAD), lambda c: (c, 0, 0))
    col = pl.BlockSpec((1, K_PAD, 1), lambda c: (c, 0, 0))
    rows3 = lambda a: a.reshape(nclass, 1, K_PAD)
    cols3 = lambda a: a.reshape(nclass, K_PAD, 1)
    kept = pl.pallas_call(
        _nms_kernel,
        grid=(nclass,),
        in_specs=[row, row, row, row, col, col, col, col, row],
        out_specs=row,
        out_shape=jax.ShapeDtypeStruct((nclass, 1, K_PAD), jnp.float32),
    )(rows3(bx1p), rows3(by1p), rows3(bx2p), rows3(by2p),
      cols3(bx1p), cols3(by1p), cols3(bx2p), cols3(by2p), rows3(scp))
    kept = kept.reshape(nclass, K_PAD)

    sc_kept = kept[:, :PRE_NMS_TOPK]
    all_scores = sc_kept.reshape(-1)
    all_boxes = jnp.stack([bx1, by1, bx2, by2], axis=-1).reshape(-1, 4)
    labels = jnp.broadcast_to(
        (jnp.arange(nclass, dtype=jnp.int32) + 1)[:, None],
        (nclass, PRE_NMS_TOPK)).reshape(-1)
    fs, find = jax.lax.top_k(all_scores, DET_PER_IMG)
    return all_boxes[find], fs, labels[find]
